# Initial kernel scaffold; baseline (speedup 1.0000x reference)
import jax
import jax.numpy as jnp
from jax.experimental import pallas as pl


def kernel(inputs, targets, pad_ratios_bidirection, part_idx, lut, cq, weight):
    raise NotImplementedError("write your pallas kernel here")


# speedup vs baseline: 1.1026x; 1.1026x over previous
import jax
import jax.numpy as jnp
from jax.experimental import pallas as pl
from jax.experimental.pallas import tpu as pltpu

_B = 1024
_F = 64
_N = 100000 + 5000
_SCALAR = 30.0
_NPART = 7
_IGNORE = 5555
_NB = 2048
_GRID = (_N + _NB - 1) // _NB


def _oim_block(x_ref, tab_ref, st_ref, w_ref, logits_ref, loss_ref,
               m_ref, s_ref, tl_ref):
    i = pl.program_id(0)
    x = x_ref[...]
    t = tab_ref[...]
    logits = jax.lax.dot_general(
        x, t, (((1,), (1,)), ((), ())),
        preferred_element_type=jnp.float32) * _SCALAR
    logits_ref[...] = logits

    col = i * _NB + jax.lax.broadcasted_iota(jnp.int32, (1, _NB), 1)
    inb = col < _N
    neg = jnp.float32(-jnp.inf)
    lm = jnp.where(inb, logits, neg)
    bm = jnp.max(lm, axis=1, keepdims=True)
    p = jnp.where(inb, jnp.exp(lm - bm), 0.0)
    bs = jnp.sum(p, axis=1, keepdims=True)
    eq = col == st_ref[...]
    btl = jnp.sum(jnp.where(eq, logits, 0.0), axis=1, keepdims=True)

    @pl.when(i == 0)
    def _init():
        m_ref[...] = jnp.full((_B, 1), neg, jnp.float32)
        s_ref[...] = jnp.zeros((_B, 1), jnp.float32)
        tl_ref[...] = jnp.zeros((_B, 1), jnp.float32)

    m_old = m_ref[...]
    m_new = jnp.maximum(m_old, bm)
    s_ref[...] = s_ref[...] * jnp.exp(m_old - m_new) + bs * jnp.exp(bm - m_new)
    m_ref[...] = m_new
    tl_ref[...] = tl_ref[...] + btl

    @pl.when(i == _GRID - 1)
    def _finish():
        lse = m_ref[...] + jnp.log(s_ref[...])
        nll = lse - tl_ref[...]
        w = w_ref[...]
        num = jnp.sum(w * nll, axis=0, keepdims=True)
        den = jnp.maximum(jnp.sum(w, axis=0, keepdims=True), 1e-12)
        loss_ref[...] = num / den


def kernel(inputs, targets, pad_ratios_bidirection, part_idx, lut, cq, weight):
    uppad_ratios = pad_ratios_bidirection[:, 0]
    pad_ratios = pad_ratios_bidirection[:, 1]
    vis_part_up = _NPART - jnp.ceil(_NPART * (1.0 - uppad_ratios))
    vis_part_down = jnp.ceil(_NPART * (1.0 - pad_ratios))
    invis = (part_idx > vis_part_down) | (part_idx <= vis_part_up)
    unlab = targets < 0
    new_targets = jnp.where(invis | unlab, _IGNORE, targets)
    valid = new_targets != _IGNORE
    safe_t = jnp.where(valid, new_targets, 0)
    w = weight[safe_t] * valid.astype(jnp.float32)

    table = jnp.concatenate([lut, cq], axis=0)
    logits, loss = pl.pallas_call(
        _oim_block,
        grid=(_GRID,),
        in_specs=[
            pl.BlockSpec((_B, _F), lambda i: (0, 0)),
            pl.BlockSpec((_NB, _F), lambda i: (i, 0)),
            pl.BlockSpec((_B, 1), lambda i: (0, 0)),
            pl.BlockSpec((_B, 1), lambda i: (0, 0)),
        ],
        out_specs=[
            pl.BlockSpec((_B, _NB), lambda i: (0, i)),
            pl.BlockSpec((1, 1), lambda i: (0, 0)),
        ],
        out_shape=[
            jax.ShapeDtypeStruct((_B, _N), jnp.float32),
            jax.ShapeDtypeStruct((1, 1), jnp.float32),
        ],
        scratch_shapes=[pltpu.VMEM((_B, 1), jnp.float32)] * 3,
    )(inputs, table, safe_t.reshape(_B, 1), w.reshape(_B, 1))
    return loss[0, 0], logits


# speedup vs baseline: 1.1288x; 1.0238x over previous
import jax
import jax.numpy as jnp
from jax.experimental import pallas as pl
from jax.experimental.pallas import tpu as pltpu

_B = 1024
_F = 64
_N = 100000 + 5000
_SCALAR = 30.0
_NPART = 7
_IGNORE = 5555
_NB = 2048
_GRID = (_N + _NB - 1) // _NB


def _oim_block(x_ref, tab_ref, tr_ref, w_ref, logits_ref, loss_ref,
               m_ref, s_ref):
    i = pl.program_id(0)
    logits_ref[...] = jax.lax.dot_general(
        x_ref[...], tab_ref[...], (((1,), (1,)), ((), ())),
        preferred_element_type=jnp.float32) * _SCALAR

    neg = jnp.float32(-jnp.inf)

    @pl.when(i == 0)
    def _init():
        m_ref[...] = jnp.full((_B, 1), neg, jnp.float32)
        s_ref[...] = jnp.zeros((_B, 1), jnp.float32)

    col = i * _NB + jax.lax.broadcasted_iota(jnp.int32, (1, _NB), 1)
    lm = jnp.where(col < _N, logits_ref[...], neg)
    bm = jnp.max(lm, axis=1, keepdims=True)
    bs = jnp.sum(jnp.exp(lm - bm), axis=1, keepdims=True)
    m_old = m_ref[...]
    m_new = jnp.maximum(m_old, bm)
    s_ref[...] = s_ref[...] * jnp.exp(m_old - m_new) + bs * jnp.exp(bm - m_new)
    m_ref[...] = m_new

    @pl.when(i == _GRID - 1)
    def _finish():
        lse = m_ref[...] + jnp.log(s_ref[...])
        tl = jnp.sum(x_ref[...] * tr_ref[...], axis=1, keepdims=True) * _SCALAR
        nll = lse - tl
        w = w_ref[...]
        num = jnp.sum(w * nll, axis=0, keepdims=True)
        den = jnp.maximum(jnp.sum(w, axis=0, keepdims=True), 1e-12)
        loss_ref[...] = num / den


def kernel(inputs, targets, pad_ratios_bidirection, part_idx, lut, cq, weight):
    uppad_ratios = pad_ratios_bidirection[:, 0]
    pad_ratios = pad_ratios_bidirection[:, 1]
    vis_part_up = _NPART - jnp.ceil(_NPART * (1.0 - uppad_ratios))
    vis_part_down = jnp.ceil(_NPART * (1.0 - pad_ratios))
    invis = (part_idx > vis_part_down) | (part_idx <= vis_part_up)
    unlab = targets < 0
    new_targets = jnp.where(invis | unlab, _IGNORE, targets)
    valid = new_targets != _IGNORE
    safe_t = jnp.where(valid, new_targets, 0)
    w = weight[safe_t] * valid.astype(jnp.float32)

    table = jnp.concatenate([lut, cq], axis=0)
    target_rows = jnp.take(table, safe_t, axis=0)
    logits, loss = pl.pallas_call(
        _oim_block,
        grid=(_GRID,),
        in_specs=[
            pl.BlockSpec((_B, _F), lambda i: (0, 0)),
            pl.BlockSpec((_NB, _F), lambda i: (i, 0)),
            pl.BlockSpec((_B, _F), lambda i: (0, 0)),
            pl.BlockSpec((_B, 1), lambda i: (0, 0)),
        ],
        out_specs=[
            pl.BlockSpec((_B, _NB), lambda i: (0, i)),
            pl.BlockSpec((1, 1), lambda i: (0, 0)),
        ],
        out_shape=[
            jax.ShapeDtypeStruct((_B, _N), jnp.float32),
            jax.ShapeDtypeStruct((1, 1), jnp.float32),
        ],
        scratch_shapes=[pltpu.VMEM((_B, 1), jnp.float32)] * 2,
        compiler_params=pltpu.CompilerParams(
            vmem_limit_bytes=100 * 1024 * 1024),
    )(inputs, table, target_rows, w.reshape(_B, 1))
    return loss[0, 0], logits


# speedup vs baseline: 2.5985x; 2.3019x over previous
import jax
import jax.numpy as jnp
from jax.experimental import pallas as pl
from jax.experimental.pallas import tpu as pltpu

_B = 1024
_F = 64
_N = 100000 + 5000
_SCALAR = 30.0
_NPART = 7
_IGNORE = 5555
_NB = 2048
_GRID = (_N + _NB - 1) // _NB


def _oim_block(xt_ref, tab_ref, trt_ref, w_ref, logits_ref, loss_ref,
               m_ref, s_ref):
    i = pl.program_id(0)
    logits_ref[...] = jax.lax.dot_general(
        tab_ref[...], xt_ref[...], (((1,), (0,)), ((), ())),
        preferred_element_type=jnp.float32) * _SCALAR

    neg = jnp.float32(-jnp.inf)

    @pl.when(i == 0)
    def _init():
        m_ref[...] = jnp.full((1, _B), neg, jnp.float32)
        s_ref[...] = jnp.zeros((1, _B), jnp.float32)

    rid = i * _NB + jax.lax.broadcasted_iota(jnp.int32, (_NB, 1), 0)
    lm = jnp.where(rid < _N, logits_ref[...], neg)
    bm = jnp.max(lm, axis=0, keepdims=True)
    bs = jnp.sum(jnp.exp(lm - bm), axis=0, keepdims=True)
    m_old = m_ref[...]
    m_new = jnp.maximum(m_old, bm)
    s_ref[...] = s_ref[...] * jnp.exp(m_old - m_new) + bs * jnp.exp(bm - m_new)
    m_ref[...] = m_new

    @pl.when(i == _GRID - 1)
    def _finish():
        lse = m_ref[...] + jnp.log(s_ref[...])
        tl = jnp.sum(xt_ref[...] * trt_ref[...], axis=0, keepdims=True) * _SCALAR
        nll = lse - tl
        w = w_ref[...]
        num = jnp.sum(w * nll, axis=1, keepdims=True)
        den = jnp.maximum(jnp.sum(w, axis=1, keepdims=True), 1e-12)
        loss_ref[...] = num / den


def kernel(inputs, targets, pad_ratios_bidirection, part_idx, lut, cq, weight):
    uppad_ratios = pad_ratios_bidirection[:, 0]
    pad_ratios = pad_ratios_bidirection[:, 1]
    vis_part_up = _NPART - jnp.ceil(_NPART * (1.0 - uppad_ratios))
    vis_part_down = jnp.ceil(_NPART * (1.0 - pad_ratios))
    invis = (part_idx > vis_part_down) | (part_idx <= vis_part_up)
    unlab = targets < 0
    new_targets = jnp.where(invis | unlab, _IGNORE, targets)
    valid = new_targets != _IGNORE
    safe_t = jnp.where(valid, new_targets, 0)
    w = weight[safe_t] * valid.astype(jnp.float32)

    table = jnp.concatenate([lut, cq], axis=0)
    xt = inputs.T
    trt = jnp.take(table, safe_t, axis=0).T
    logits_t, loss = pl.pallas_call(
        _oim_block,
        grid=(_GRID,),
        in_specs=[
            pl.BlockSpec((_F, _B), lambda i: (0, 0)),
            pl.BlockSpec((_NB, _F), lambda i: (i, 0)),
            pl.BlockSpec((_F, _B), lambda i: (0, 0)),
            pl.BlockSpec((1, _B), lambda i: (0, 0)),
        ],
        out_specs=[
            pl.BlockSpec((_NB, _B), lambda i: (i, 0)),
            pl.BlockSpec((1, 1), lambda i: (0, 0)),
        ],
        out_shape=[
            jax.ShapeDtypeStruct((_N, _B), jnp.float32),
            jax.ShapeDtypeStruct((1, 1), jnp.float32),
        ],
        scratch_shapes=[pltpu.VMEM((1, _B), jnp.float32)] * 2,
        compiler_params=pltpu.CompilerParams(
            vmem_limit_bytes=100 * 1024 * 1024),
    )(xt, table, trt, w.reshape(1, _B))
    return loss[0, 0], jnp.transpose(logits_t)
